# baseline (speedup 1.0000x reference)
import functools

import jax
import jax.numpy as jnp
from jax import lax
from jax.experimental import pallas as pl
from jax.experimental.pallas import tpu as pltpu
from jax.experimental.pallas import tpu_sc as plsc

B = 16384
ED = 64
FD = 32
HD = 128
NC = 2
NS = 16
NW = NC * NS
RPW = B // NW
CHUNK = 128
NCH = RPW // CHUNK
TB = 2048


def _sc_gather_body(uidx, iidx, f1idx, f2idx, g1idx,
                    uet, iet, u1t, u2t, i1t,
                    out_ue, out_ie, out_f1, out_f2, out_g1,
                    uiv, iiv, f1v, f2v, g1v,
                    uev, iev, f1r, f2r, g1r,
                    sem0, sem1, sem2, sem3, sem4):
    wid = lax.axis_index("s") * NC + lax.axis_index("c")
    crow = wid * NCH
    pltpu.sync_copy(uidx.at[pl.ds(crow, NCH)], uiv)
    pltpu.sync_copy(iidx.at[pl.ds(crow, NCH)], iiv)
    pltpu.sync_copy(f1idx.at[pl.ds(crow, NCH)], f1v)
    pltpu.sync_copy(f2idx.at[pl.ds(crow, NCH)], f2v)
    pltpu.sync_copy(g1idx.at[pl.ds(crow, NCH)], g1v)
    for j in range(NCH):
        base = wid * RPW + j * CHUNK
        handles = [
            pltpu.async_copy(uet.at[uiv.at[j]], uev, sem0),
            pltpu.async_copy(iet.at[iiv.at[j]], iev, sem1),
            pltpu.async_copy(u1t.at[f1v.at[j]], f1r, sem2),
            pltpu.async_copy(u2t.at[f2v.at[j]], f2r, sem3),
            pltpu.async_copy(i1t.at[g1v.at[j]], g1r, sem4),
        ]
        for h in handles:
            h.wait()
        pltpu.sync_copy(uev, out_ue.at[pl.ds(base, CHUNK)])
        pltpu.sync_copy(iev, out_ie.at[pl.ds(base, CHUNK)])
        pltpu.sync_copy(f1r, out_f1.at[pl.ds(base, CHUNK)])
        pltpu.sync_copy(f2r, out_f2.at[pl.ds(base, CHUNK)])
        pltpu.sync_copy(g1r, out_g1.at[pl.ds(base, CHUNK)])


_sc_gather = functools.partial(
    pl.kernel,
    mesh=plsc.VectorSubcoreMesh(core_axis_name="c", subcore_axis_name="s"),
    out_type=[
        jax.ShapeDtypeStruct((B, 128), jnp.float32),
        jax.ShapeDtypeStruct((B, 128), jnp.float32),
        jax.ShapeDtypeStruct((B, 128), jnp.float32),
        jax.ShapeDtypeStruct((B, 128), jnp.float32),
        jax.ShapeDtypeStruct((B, 128), jnp.float32),
    ],
    scratch_types=[
        pltpu.VMEM((NCH, CHUNK), jnp.int32),
        pltpu.VMEM((NCH, CHUNK), jnp.int32),
        pltpu.VMEM((NCH, CHUNK), jnp.int32),
        pltpu.VMEM((NCH, CHUNK), jnp.int32),
        pltpu.VMEM((NCH, CHUNK), jnp.int32),
        pltpu.VMEM((CHUNK, 128), jnp.float32),
        pltpu.VMEM((CHUNK, 128), jnp.float32),
        pltpu.VMEM((CHUNK, 128), jnp.float32),
        pltpu.VMEM((CHUNK, 128), jnp.float32),
        pltpu.VMEM((CHUNK, 128), jnp.float32),
        pltpu.SemaphoreType.DMA,
        pltpu.SemaphoreType.DMA,
        pltpu.SemaphoreType.DMA,
        pltpu.SemaphoreType.DMA,
        pltpu.SemaphoreType.DMA,
    ],
)(_sc_gather_body)


def _sel2(packed, hcol, w):
    lo = packed[:, :w]
    hi = packed[:, w:2 * w]
    m = (hcol == 1).astype(jnp.float32)
    return lo + (hi - lo) * m


def _sel4(packed, hcol, w):
    acc = None
    for q in range(4):
        m = (hcol == q).astype(jnp.float32)
        term = packed[:, q * w:(q + 1) * w] * m
        acc = term if acc is None else acc + term
    return acc


def _tower_body(ues, ies, f1s, f2s, g1s, hsel, uf, itf,
                w1ua, w1ub, w1uc, w1ud, b1u, w2u, b2u,
                w1ia, w1ib, w1ic, b1i, w2i, b2i, out):
    h = hsel[:]
    ue = _sel2(ues[:], h[:, 0:1], ED)
    ie = _sel2(ies[:], h[:, 1:2], ED)
    f1 = _sel4(f1s[:], h[:, 2:3], FD)
    f2 = _sel4(f2s[:], h[:, 3:4], FD)
    g1 = _sel4(g1s[:], h[:, 4:5], FD)

    xu = jnp.dot(ue, w1ua[:], preferred_element_type=jnp.float32)
    xu += jnp.dot(f1, w1ub[:], preferred_element_type=jnp.float32)
    xu += jnp.dot(f2, w1uc[:], preferred_element_type=jnp.float32)
    xu += uf[:][:, 2:3] * w1ud[:] + b1u[:]
    xu = jnp.maximum(xu, 0.0)
    ur = jnp.dot(xu, w2u[:], preferred_element_type=jnp.float32) + b2u[:]

    yi = jnp.dot(ie, w1ia[:], preferred_element_type=jnp.float32)
    yi += jnp.dot(g1, w1ib[:], preferred_element_type=jnp.float32)
    yi += jnp.dot(itf[:][:, 1:18], w1ic[:], preferred_element_type=jnp.float32)
    yi = jnp.maximum(yi + b1i[:], 0.0)
    ir = jnp.dot(yi, w2i[:], preferred_element_type=jnp.float32) + b2i[:]

    out[0, 0, :] = jnp.sum(ur * ir, axis=1)


def kernel(user_ids, item_ids, user_features, item_features, user_emb,
           item_emb, u_cat1, u_cat2, i_cat1, user_fc1_W, user_fc1_b,
           user_fc2_W, user_fc2_b, item_fc1_W, item_fc1_b, item_fc2_W,
           item_fc2_b):
    uid = user_ids.astype(jnp.int32)
    iid = item_ids.astype(jnp.int32)
    f1i = user_features[:, 0].astype(jnp.int32)
    f2i = user_features[:, 1].astype(jnp.int32)
    g1i = item_features[:, 0].astype(jnp.int32)

    uidx = (uid >> 1).reshape(NW * NCH, CHUNK)
    iidx = (iid >> 1).reshape(NW * NCH, CHUNK)
    f1idx = (f1i >> 2).reshape(NW * NCH, CHUNK)
    f2idx = (f2i >> 2).reshape(NW * NCH, CHUNK)
    g1idx = (g1i >> 2).reshape(NW * NCH, CHUNK)
    hsel = jnp.stack([uid & 1, iid & 1, f1i & 3, f2i & 3, g1i & 3], axis=1)

    uet = user_emb.reshape(-1, 128)
    iet = item_emb.reshape(-1, 128)
    u1t = u_cat1.reshape(-1, 128)
    u2t = u_cat2.reshape(-1, 128)
    i1t = i_cat1.reshape(-1, 128)

    ues, ies, f1s, f2s, g1s = _sc_gather(uidx, iidx, f1idx, f2idx, g1idx,
                                         uet, iet, u1t, u2t, i1t)

    grid = B // TB
    scores = pl.pallas_call(
        _tower_body,
        grid=(grid,),
        in_specs=[
            pl.BlockSpec((TB, 128), lambda i: (i, 0)),
            pl.BlockSpec((TB, 128), lambda i: (i, 0)),
            pl.BlockSpec((TB, 128), lambda i: (i, 0)),
            pl.BlockSpec((TB, 128), lambda i: (i, 0)),
            pl.BlockSpec((TB, 128), lambda i: (i, 0)),
            pl.BlockSpec((TB, 5), lambda i: (i, 0)),
            pl.BlockSpec((TB, 3), lambda i: (i, 0)),
            pl.BlockSpec((TB, 18), lambda i: (i, 0)),
            pl.BlockSpec((ED, HD), lambda i: (0, 0)),
            pl.BlockSpec((FD, HD), lambda i: (0, 0)),
            pl.BlockSpec((FD, HD), lambda i: (0, 0)),
            pl.BlockSpec((1, HD), lambda i: (0, 0)),
            pl.BlockSpec((1, HD), lambda i: (0, 0)),
            pl.BlockSpec((HD, HD), lambda i: (0, 0)),
            pl.BlockSpec((1, HD), lambda i: (0, 0)),
            pl.BlockSpec((ED, HD), lambda i: (0, 0)),
            pl.BlockSpec((FD, HD), lambda i: (0, 0)),
            pl.BlockSpec((17, HD), lambda i: (0, 0)),
            pl.BlockSpec((1, HD), lambda i: (0, 0)),
            pl.BlockSpec((HD, HD), lambda i: (0, 0)),
            pl.BlockSpec((1, HD), lambda i: (0, 0)),
        ],
        out_specs=pl.BlockSpec((1, 1, TB), lambda i: (i, 0, 0)),
        out_shape=jax.ShapeDtypeStruct((grid, 1, TB), jnp.float32),
    )(
        ues, ies, f1s, f2s, g1s, hsel, user_features, item_features,
        user_fc1_W[:ED], user_fc1_W[ED:ED + FD], user_fc1_W[ED + FD:128],
        user_fc1_W[128:129], user_fc1_b.reshape(1, HD),
        user_fc2_W, user_fc2_b.reshape(1, HD),
        item_fc1_W[:ED], item_fc1_W[ED:ED + FD], item_fc1_W[ED + FD:113],
        item_fc1_b.reshape(1, HD),
        item_fc2_W, item_fc2_b.reshape(1, HD),
    )
    return scores.reshape(B)


# speedup vs baseline: 1.3712x; 1.3712x over previous
import functools

import jax
import jax.numpy as jnp
from jax import lax
from jax.experimental import pallas as pl
from jax.experimental.pallas import tpu as pltpu
from jax.experimental.pallas import tpu_sc as plsc

B = 16384
ED = 64
FD = 32
HD = 128
NC = 2
NS = 16
NW = NC * NS
RPW = B // NW
CHUNK = 128
NCH = RPW // CHUNK
RC = 16
NRC = RPW // RC
TB = 2048


def _sc_small_body(f1idx, f2idx, g1idx, u1t, u2t, i1t,
                   out_f1, out_f2, out_g1,
                   f1v, f2v, g1v, f1r, f2r, g1r,
                   sem2, sem3, sem4):
    wid = lax.axis_index("s") * NC + lax.axis_index("c")
    crow = wid * NCH
    base = wid * RPW
    pltpu.sync_copy(f1idx.at[pl.ds(crow, NCH)], f1v)
    pltpu.sync_copy(f2idx.at[pl.ds(crow, NCH)], f2v)
    pltpu.sync_copy(g1idx.at[pl.ds(crow, NCH)], g1v)
    handles = []
    for j in range(NCH):
        o = j * CHUNK
        handles.append(pltpu.async_copy(u1t.at[f1v.at[j]], f1r.at[pl.ds(o, CHUNK)], sem2))
        handles.append(pltpu.async_copy(u2t.at[f2v.at[j]], f2r.at[pl.ds(o, CHUNK)], sem3))
        handles.append(pltpu.async_copy(i1t.at[g1v.at[j]], g1r.at[pl.ds(o, CHUNK)], sem4))
    for h in handles:
        h.wait()
    pltpu.sync_copy(f1r, out_f1.at[pl.ds(base, RPW)])
    pltpu.sync_copy(f2r, out_f2.at[pl.ds(base, RPW)])
    pltpu.sync_copy(g1r, out_g1.at[pl.ds(base, RPW)])


_sc_small = functools.partial(
    pl.kernel,
    mesh=plsc.VectorSubcoreMesh(core_axis_name="c", subcore_axis_name="s"),
    compiler_params=pltpu.CompilerParams(use_tc_tiling_on_sc=False),
    out_type=[
        jax.ShapeDtypeStruct((B, FD), jnp.float32),
        jax.ShapeDtypeStruct((B, FD), jnp.float32),
        jax.ShapeDtypeStruct((B, FD), jnp.float32),
    ],
    scratch_types=[
        pltpu.VMEM((NCH, CHUNK), jnp.int32),
        pltpu.VMEM((NCH, CHUNK), jnp.int32),
        pltpu.VMEM((NCH, CHUNK), jnp.int32),
        pltpu.VMEM((RPW, FD), jnp.float32),
        pltpu.VMEM((RPW, FD), jnp.float32),
        pltpu.VMEM((RPW, FD), jnp.float32),
        pltpu.SemaphoreType.DMA,
        pltpu.SemaphoreType.DMA,
        pltpu.SemaphoreType.DMA,
    ],
)(_sc_small_body)


def _sc_slab_body(idx, table, out, ivm, slab, rows, sem):
    wid = lax.axis_index("s") * NC + lax.axis_index("c")
    pltpu.sync_copy(idx.at[wid], ivm)

    def chunk(c, carry):
        cbase = pl.multiple_of(c * RC, RC)
        vec = ivm[pl.ds(cbase, RC)]
        handles = []
        for k in range(RC):
            s = vec[k]
            sb = pl.multiple_of((s >> 3) * 8, 8)
            handles.append(pltpu.async_copy(
                table.at[pl.ds(sb, 8)], slab.at[pl.ds(k * 8, 8)], sem))
        for h in handles:
            h.wait()
        for k in range(RC):
            h = (vec[k] & 7) + k * 8
            for j in range(ED // 16):
                rows[k, pl.ds(j * 16, 16)] = slab[h, pl.ds(j * 16, 16)]
        obase = pl.multiple_of(wid * RPW + c * RC, RC)
        pltpu.sync_copy(rows, out.at[pl.ds(obase, RC)])
        return carry

    lax.fori_loop(0, NRC, chunk, 0)


_sc_slab = functools.partial(
    pl.kernel,
    mesh=plsc.VectorSubcoreMesh(core_axis_name="c", subcore_axis_name="s"),
    out_type=jax.ShapeDtypeStruct((B, ED), jnp.float32),
    scratch_types=[
        pltpu.VMEM((RPW,), jnp.int32),
        pltpu.VMEM((RC * 8, ED), jnp.float32),
        pltpu.VMEM((RC, ED), jnp.float32),
        pltpu.SemaphoreType.DMA,
    ],
)(_sc_slab_body)


def _tower_body(ue, ie, f1, f2, g1, uf, itf,
                w1ua, w1ub, w1uc, w1ud, b1u, w2u, b2u,
                w1ia, w1ib, w1ic, b1i, w2i, b2i, out):
    xu = jnp.dot(ue[:], w1ua[:], preferred_element_type=jnp.float32)
    xu += jnp.dot(f1[:], w1ub[:], preferred_element_type=jnp.float32)
    xu += jnp.dot(f2[:], w1uc[:], preferred_element_type=jnp.float32)
    xu += uf[:][:, 2:3] * w1ud[:] + b1u[:]
    xu = jnp.maximum(xu, 0.0)
    ur = jnp.dot(xu, w2u[:], preferred_element_type=jnp.float32) + b2u[:]

    yi = jnp.dot(ie[:], w1ia[:], preferred_element_type=jnp.float32)
    yi += jnp.dot(g1[:], w1ib[:], preferred_element_type=jnp.float32)
    yi += jnp.dot(itf[:][:, 1:18], w1ic[:], preferred_element_type=jnp.float32)
    yi = jnp.maximum(yi + b1i[:], 0.0)
    ir = jnp.dot(yi, w2i[:], preferred_element_type=jnp.float32) + b2i[:]

    out[0, 0, :] = jnp.sum(ur * ir, axis=1)


def kernel(user_ids, item_ids, user_features, item_features, user_emb,
           item_emb, u_cat1, u_cat2, i_cat1, user_fc1_W, user_fc1_b,
           user_fc2_W, user_fc2_b, item_fc1_W, item_fc1_b, item_fc2_W,
           item_fc2_b):
    uidx = user_ids.astype(jnp.int32).reshape(NW, RPW)
    iidx = item_ids.astype(jnp.int32).reshape(NW, RPW)
    f1idx = user_features[:, 0].astype(jnp.int32).reshape(NW * NCH, CHUNK)
    f2idx = user_features[:, 1].astype(jnp.int32).reshape(NW * NCH, CHUNK)
    g1idx = item_features[:, 0].astype(jnp.int32).reshape(NW * NCH, CHUNK)

    f1, f2, g1 = _sc_small(f1idx, f2idx, g1idx, u_cat1, u_cat2, i_cat1)
    ue = _sc_slab(uidx, user_emb)
    ie = _sc_slab(iidx, item_emb)

    grid = B // TB
    scores = pl.pallas_call(
        _tower_body,
        grid=(grid,),
        in_specs=[
            pl.BlockSpec((TB, ED), lambda i: (i, 0)),
            pl.BlockSpec((TB, ED), lambda i: (i, 0)),
            pl.BlockSpec((TB, FD), lambda i: (i, 0)),
            pl.BlockSpec((TB, FD), lambda i: (i, 0)),
            pl.BlockSpec((TB, FD), lambda i: (i, 0)),
            pl.BlockSpec((TB, 3), lambda i: (i, 0)),
            pl.BlockSpec((TB, 18), lambda i: (i, 0)),
            pl.BlockSpec((ED, HD), lambda i: (0, 0)),
            pl.BlockSpec((FD, HD), lambda i: (0, 0)),
            pl.BlockSpec((FD, HD), lambda i: (0, 0)),
            pl.BlockSpec((1, HD), lambda i: (0, 0)),
            pl.BlockSpec((1, HD), lambda i: (0, 0)),
            pl.BlockSpec((HD, HD), lambda i: (0, 0)),
            pl.BlockSpec((1, HD), lambda i: (0, 0)),
            pl.BlockSpec((ED, HD), lambda i: (0, 0)),
            pl.BlockSpec((FD, HD), lambda i: (0, 0)),
            pl.BlockSpec((17, HD), lambda i: (0, 0)),
            pl.BlockSpec((1, HD), lambda i: (0, 0)),
            pl.BlockSpec((HD, HD), lambda i: (0, 0)),
            pl.BlockSpec((1, HD), lambda i: (0, 0)),
        ],
        out_specs=pl.BlockSpec((1, 1, TB), lambda i: (i, 0, 0)),
        out_shape=jax.ShapeDtypeStruct((grid, 1, TB), jnp.float32),
    )(
        ue, ie, f1, f2, g1, user_features, item_features,
        user_fc1_W[:ED], user_fc1_W[ED:ED + FD], user_fc1_W[ED + FD:128],
        user_fc1_W[128:129], user_fc1_b.reshape(1, HD),
        user_fc2_W, user_fc2_b.reshape(1, HD),
        item_fc1_W[:ED], item_fc1_W[ED:ED + FD], item_fc1_W[ED + FD:113],
        item_fc1_b.reshape(1, HD),
        item_fc2_W, item_fc2_b.reshape(1, HD),
    )
    return scores.reshape(B)


# speedup vs baseline: 1.3872x; 1.0117x over previous
import functools

import jax
import jax.numpy as jnp
from jax import lax
from jax.experimental import pallas as pl
from jax.experimental.pallas import tpu as pltpu
from jax.experimental.pallas import tpu_sc as plsc

B = 16384
ED = 64
FD = 32
HD = 128
NC = 2
NS = 16
NW = NC * NS
RPW = B // NW
CHUNK = 128
NCH = RPW // CHUNK
RC = 16
NRC = RPW // RC
TB = 2048


def _sc_small_body(f1idx, f2idx, g1idx, u1t, u2t, i1t,
                   out_f1, out_f2, out_g1,
                   f1v, f2v, g1v, f1r, f2r, g1r,
                   sem2, sem3, sem4):
    wid = lax.axis_index("s") * NC + lax.axis_index("c")
    crow = wid * NCH
    base = wid * RPW
    pltpu.sync_copy(f1idx.at[pl.ds(crow, NCH)], f1v)
    pltpu.sync_copy(f2idx.at[pl.ds(crow, NCH)], f2v)
    pltpu.sync_copy(g1idx.at[pl.ds(crow, NCH)], g1v)
    handles = []
    for j in range(NCH):
        o = j * CHUNK
        handles.append(pltpu.async_copy(u1t.at[f1v.at[j]], f1r.at[pl.ds(o, CHUNK)], sem2))
        handles.append(pltpu.async_copy(u2t.at[f2v.at[j]], f2r.at[pl.ds(o, CHUNK)], sem3))
        handles.append(pltpu.async_copy(i1t.at[g1v.at[j]], g1r.at[pl.ds(o, CHUNK)], sem4))
    for h in handles:
        h.wait()
    pltpu.sync_copy(f1r, out_f1.at[pl.ds(base, RPW)])
    pltpu.sync_copy(f2r, out_f2.at[pl.ds(base, RPW)])
    pltpu.sync_copy(g1r, out_g1.at[pl.ds(base, RPW)])


_sc_small = functools.partial(
    pl.kernel,
    mesh=plsc.VectorSubcoreMesh(core_axis_name="c", subcore_axis_name="s"),
    compiler_params=pltpu.CompilerParams(use_tc_tiling_on_sc=False),
    out_type=[
        jax.ShapeDtypeStruct((B, FD), jnp.float32),
        jax.ShapeDtypeStruct((B, FD), jnp.float32),
        jax.ShapeDtypeStruct((B, FD), jnp.float32),
    ],
    scratch_types=[
        pltpu.VMEM((NCH, CHUNK), jnp.int32),
        pltpu.VMEM((NCH, CHUNK), jnp.int32),
        pltpu.VMEM((NCH, CHUNK), jnp.int32),
        pltpu.VMEM((RPW, FD), jnp.float32),
        pltpu.VMEM((RPW, FD), jnp.float32),
        pltpu.VMEM((RPW, FD), jnp.float32),
        pltpu.SemaphoreType.DMA,
        pltpu.SemaphoreType.DMA,
        pltpu.SemaphoreType.DMA,
    ],
)(_sc_small_body)


def _sc_slab_body(idx, table, out, ivm, slab_a, slab_b, rows_a, rows_b,
                  sem_a, sem_b):
    wid = lax.axis_index("s") * NC + lax.axis_index("c")
    pltpu.sync_copy(idx.at[wid], ivm)

    def fire(vec, slab, sem):
        handles = []
        for k in range(RC):
            s = vec[k]
            sb = pl.multiple_of((s >> 3) * 8, 8)
            handles.append(pltpu.async_copy(
                table.at[pl.ds(sb, 8)], slab.at[pl.ds(k * 8, 8)], sem))
        return handles

    def drain_select_store(vec, slab, rows, handles, c):
        for h in handles:
            h.wait()
        for k in range(RC):
            h = (vec[k] & 7) + k * 8
            for j in range(ED // 16):
                rows[k, pl.ds(j * 16, 16)] = slab[h, pl.ds(j * 16, 16)]
        obase = pl.multiple_of(wid * RPW + c * RC, RC)
        pltpu.sync_copy(rows, out.at[pl.ds(obase, RC)])

    def chunk2(c2, carry):
        ca = pl.multiple_of(c2 * 2, 2)
        cb = ca + 1
        vec_a = ivm[pl.ds(pl.multiple_of(ca * RC, RC), RC)]
        vec_b = ivm[pl.ds(pl.multiple_of(cb * RC, RC), RC)]
        ha = fire(vec_a, slab_a, sem_a)
        hb = fire(vec_b, slab_b, sem_b)
        drain_select_store(vec_a, slab_a, rows_a, ha, ca)
        drain_select_store(vec_b, slab_b, rows_b, hb, cb)
        return carry

    lax.fori_loop(0, NRC // 2, chunk2, 0)


_sc_slab = functools.partial(
    pl.kernel,
    mesh=plsc.VectorSubcoreMesh(core_axis_name="c", subcore_axis_name="s"),
    out_type=jax.ShapeDtypeStruct((B, ED), jnp.float32),
    scratch_types=[
        pltpu.VMEM((RPW,), jnp.int32),
        pltpu.VMEM((RC * 8, ED), jnp.float32),
        pltpu.VMEM((RC * 8, ED), jnp.float32),
        pltpu.VMEM((RC, ED), jnp.float32),
        pltpu.VMEM((RC, ED), jnp.float32),
        pltpu.SemaphoreType.DMA,
        pltpu.SemaphoreType.DMA,
    ],
)(_sc_slab_body)


def _tower_body(ue, ie, f1, f2, g1, uf, itf,
                w1ua, w1ub, w1uc, w1ud, b1u, w2u, b2u,
                w1ia, w1ib, w1ic, b1i, w2i, b2i, out):
    xu = jnp.dot(ue[:], w1ua[:], preferred_element_type=jnp.float32)
    xu += jnp.dot(f1[:], w1ub[:], preferred_element_type=jnp.float32)
    xu += jnp.dot(f2[:], w1uc[:], preferred_element_type=jnp.float32)
    xu += uf[:][:, 2:3] * w1ud[:] + b1u[:]
    xu = jnp.maximum(xu, 0.0)
    ur = jnp.dot(xu, w2u[:], preferred_element_type=jnp.float32) + b2u[:]

    yi = jnp.dot(ie[:], w1ia[:], preferred_element_type=jnp.float32)
    yi += jnp.dot(g1[:], w1ib[:], preferred_element_type=jnp.float32)
    yi += jnp.dot(itf[:][:, 1:18], w1ic[:], preferred_element_type=jnp.float32)
    yi = jnp.maximum(yi + b1i[:], 0.0)
    ir = jnp.dot(yi, w2i[:], preferred_element_type=jnp.float32) + b2i[:]

    out[0, 0, :] = jnp.sum(ur * ir, axis=1)


def kernel(user_ids, item_ids, user_features, item_features, user_emb,
           item_emb, u_cat1, u_cat2, i_cat1, user_fc1_W, user_fc1_b,
           user_fc2_W, user_fc2_b, item_fc1_W, item_fc1_b, item_fc2_W,
           item_fc2_b):
    uidx = user_ids.astype(jnp.int32).reshape(NW, RPW)
    iidx = item_ids.astype(jnp.int32).reshape(NW, RPW)
    f1idx = user_features[:, 0].astype(jnp.int32).reshape(NW * NCH, CHUNK)
    f2idx = user_features[:, 1].astype(jnp.int32).reshape(NW * NCH, CHUNK)
    g1idx = item_features[:, 0].astype(jnp.int32).reshape(NW * NCH, CHUNK)

    f1, f2, g1 = _sc_small(f1idx, f2idx, g1idx, u_cat1, u_cat2, i_cat1)
    ue = _sc_slab(uidx, user_emb)
    ie = _sc_slab(iidx, item_emb)

    grid = B // TB
    scores = pl.pallas_call(
        _tower_body,
        grid=(grid,),
        in_specs=[
            pl.BlockSpec((TB, ED), lambda i: (i, 0)),
            pl.BlockSpec((TB, ED), lambda i: (i, 0)),
            pl.BlockSpec((TB, FD), lambda i: (i, 0)),
            pl.BlockSpec((TB, FD), lambda i: (i, 0)),
            pl.BlockSpec((TB, FD), lambda i: (i, 0)),
            pl.BlockSpec((TB, 3), lambda i: (i, 0)),
            pl.BlockSpec((TB, 18), lambda i: (i, 0)),
            pl.BlockSpec((ED, HD), lambda i: (0, 0)),
            pl.BlockSpec((FD, HD), lambda i: (0, 0)),
            pl.BlockSpec((FD, HD), lambda i: (0, 0)),
            pl.BlockSpec((1, HD), lambda i: (0, 0)),
            pl.BlockSpec((1, HD), lambda i: (0, 0)),
            pl.BlockSpec((HD, HD), lambda i: (0, 0)),
            pl.BlockSpec((1, HD), lambda i: (0, 0)),
            pl.BlockSpec((ED, HD), lambda i: (0, 0)),
            pl.BlockSpec((FD, HD), lambda i: (0, 0)),
            pl.BlockSpec((17, HD), lambda i: (0, 0)),
            pl.BlockSpec((1, HD), lambda i: (0, 0)),
            pl.BlockSpec((HD, HD), lambda i: (0, 0)),
            pl.BlockSpec((1, HD), lambda i: (0, 0)),
        ],
        out_specs=pl.BlockSpec((1, 1, TB), lambda i: (i, 0, 0)),
        out_shape=jax.ShapeDtypeStruct((grid, 1, TB), jnp.float32),
    )(
        ue, ie, f1, f2, g1, user_features, item_features,
        user_fc1_W[:ED], user_fc1_W[ED:ED + FD], user_fc1_W[ED + FD:128],
        user_fc1_W[128:129], user_fc1_b.reshape(1, HD),
        user_fc2_W, user_fc2_b.reshape(1, HD),
        item_fc1_W[:ED], item_fc1_W[ED:ED + FD], item_fc1_W[ED + FD:113],
        item_fc1_b.reshape(1, HD),
        item_fc2_W, item_fc2_b.reshape(1, HD),
    )
    return scores.reshape(B)
